# baseline (speedup 1.0000x reference)
import jax
import jax.numpy as jnp
from jax.experimental import pallas as pl
from jax.experimental.pallas import tpu as pltpu

_TOKENS = 32768
_DIM = 768
_EXPERTS = 64
_BT = 4096
_ROWS2 = _TOKENS * _DIM // 128
_BR = _BT * _DIM // 128


def _probe_kernel(x_ref, c_ref, t_ref, out_ref):
    out_ref[:] = x_ref[:_BT, :_EXPERTS] * t_ref[0]


@jax.jit
def kernel(x, centroids, temperature):
    x2 = x.reshape(_ROWS2, 128)
    grid = (_TOKENS // _BT,)
    return pl.pallas_call(
        _probe_kernel,
        grid=grid,
        in_specs=[
            pl.BlockSpec((_BR, 128), lambda i: (i, 0)),
            pl.BlockSpec((_EXPERTS, _DIM), lambda i: (0, 0)),
            pl.BlockSpec(memory_space=pltpu.SMEM),
        ],
        out_specs=pl.BlockSpec((_BT, _EXPERTS), lambda i: (i, 0)),
        out_shape=jax.ShapeDtypeStruct((_TOKENS, _EXPERTS), jnp.float32),
        compiler_params=pltpu.CompilerParams(
            dimension_semantics=("arbitrary",),
        ),
    )(x2, centroids, temperature)


# speedup vs baseline: 2.8224x; 2.8224x over previous
import jax
import jax.numpy as jnp
from jax.experimental import pallas as pl
from jax.experimental.pallas import tpu as pltpu

_TOKENS = 32768
_DIM = 768
_EXPERTS = 64
_CHUNK = 1024
_NCHUNK = _TOKENS // _CHUNK
_KIN = 8
_KOUT = 4


def _router_kernel(x_ref, c_ref, t_ref, out_ref,
                   xbuf, obuf, insem, outsem):
    c = c_ref[:]
    c_ss = jnp.sum(c * c, axis=1, keepdims=True)
    cn = c * jax.lax.rsqrt(jnp.maximum(c_ss, 1e-24))
    inv_t = 1.0 / t_ref[0]

    def in_copy(j):
        return pltpu.make_async_copy(
            x_ref.at[pl.ds(j * _CHUNK, _CHUNK), :],
            xbuf.at[j % _KIN],
            insem.at[j % _KIN],
        )

    def out_copy(j):
        return pltpu.make_async_copy(
            obuf.at[j % _KOUT],
            out_ref.at[pl.ds(j * _CHUNK, _CHUNK), :],
            outsem.at[j % _KOUT],
        )

    for j in range(_KIN):
        in_copy(j).start()

    for j in range(_NCHUNK):
        in_copy(j).wait()
        xb = xbuf[j % _KIN]
        x_ss = jnp.sum(xb * xb, axis=1, keepdims=True)
        inv_norm = jax.lax.rsqrt(jnp.maximum(x_ss, 1e-24))
        logits = jax.lax.dot_general(
            xb, cn, (((1,), (1,)), ((), ())), preferred_element_type=jnp.float32
        )
        if j >= _KOUT:
            out_copy(j - _KOUT).wait()
        obuf[j % _KOUT] = logits * (inv_norm * inv_t)
        out_copy(j).start()
        if j + _KIN < _NCHUNK:
            in_copy(j + _KIN).start()

    for j in range(_NCHUNK - _KOUT, _NCHUNK):
        out_copy(j).wait()


@jax.jit
def kernel(x, centroids, temperature):
    return pl.pallas_call(
        _router_kernel,
        in_specs=[
            pl.BlockSpec(memory_space=pltpu.HBM),
            pl.BlockSpec(memory_space=pltpu.VMEM),
            pl.BlockSpec(memory_space=pltpu.SMEM),
        ],
        out_specs=pl.BlockSpec(memory_space=pltpu.HBM),
        out_shape=jax.ShapeDtypeStruct((_TOKENS, _EXPERTS), jnp.float32),
        scratch_shapes=[
            pltpu.VMEM((_KIN, _CHUNK, _DIM), jnp.float32),
            pltpu.VMEM((_KOUT, _CHUNK, _EXPERTS), jnp.float32),
            pltpu.SemaphoreType.DMA((_KIN,)),
            pltpu.SemaphoreType.DMA((_KOUT,)),
        ],
    )(x, centroids, temperature)
